# baseline (speedup 1.0000x reference)
import functools

import jax
import jax.numpy as jnp
from jax import lax
from jax.experimental import pallas as pl
from jax.experimental.pallas import tpu as pltpu
from jax.experimental.pallas import tpu_sc as plsc

E = 64
TOPK = 2
D = 768
DFF = 768
T = 2048

BM = 64
S = 8192
NT = S // BM
NW = 32
BPW = 2 * T // NW
TPW = T // NW


def _csum_sublane(a, n):
    s = 1
    while s < n:
        z = jnp.zeros((s, a.shape[1]), a.dtype)
        a = a + jnp.concatenate([z, a[: n - s]], axis=0)
        s *= 2
    return a


def _route_body(x_ref, gw_ref, slots_ref, wb_ref, meta_ref):
    logits = jnp.dot(x_ref[...], gw_ref[...], preferred_element_type=jnp.float32)
    e_iota = lax.broadcasted_iota(jnp.int32, (T, E), 1)
    m1 = jnp.max(logits, axis=1, keepdims=True)
    a1 = jnp.min(jnp.where(logits == m1, e_iota, E), axis=1, keepdims=True)
    sel1 = e_iota == a1
    oh1 = sel1.astype(jnp.float32)
    masked = jnp.where(sel1, -jnp.inf, logits)
    m2 = jnp.max(masked, axis=1, keepdims=True)
    a2 = jnp.min(jnp.where(masked == m2, e_iota, E), axis=1, keepdims=True)
    oh2 = (e_iota == a2).astype(jnp.float32)
    w1 = 1.0 / (1.0 + jnp.exp(m2 - m1))
    w2 = 1.0 - w1

    csum12 = _csum_sublane(jnp.concatenate([oh1, oh2], axis=1), T)
    csum1 = csum12[:, :E]
    csum2 = csum12[:, E:]
    cnt1 = csum1[T - 1 : T, :]
    cnt2 = csum2[T - 1 : T, :]
    rank1 = jnp.sum((csum1 - oh1) * oh1, axis=1, keepdims=True)
    rank2 = jnp.sum((cnt1 + csum2 - oh2) * oh2, axis=1, keepdims=True)

    cnt_row = (cnt1 + cnt2).astype(jnp.int32)
    ee_r = lax.broadcasted_iota(jnp.int32, (E, E), 0)
    ee_c = lax.broadcasted_iota(jnp.int32, (E, E), 1)
    ident = (ee_r == ee_c).astype(jnp.int32)
    cnt_col = jnp.sum(ident * cnt_row, axis=1, keepdims=True)
    padded_col = jnp.bitwise_and(cnt_col + (BM - 1), -BM)
    incl_col = _csum_sublane(padded_col, E)
    off_col = incl_col - padded_col
    off_row = jnp.sum(ident.astype(jnp.float32) * off_col.astype(jnp.float32),
                      axis=0, keepdims=True)

    slot1 = jnp.sum(oh1 * off_row, axis=1, keepdims=True) + rank1
    slot2 = jnp.sum(oh2 * off_row, axis=1, keepdims=True) + rank2
    slots_col = jnp.concatenate([slot1, slot2], axis=0).astype(jnp.int32)
    slots_ref[...] = jnp.reshape(slots_col, (2 * T // 128, 128))
    wb_ref[...] = jnp.broadcast_to(jnp.concatenate([w1, w2], axis=0), (2 * T, 128))

    jm = lax.broadcasted_iota(jnp.int32, (E, NT), 1) * BM
    te = jnp.sum((incl_col <= jm).astype(jnp.int32), axis=0, keepdims=True)
    te = jnp.minimum(te, E - 1)
    used = lax.shift_right_logical(incl_col[E - 1 : E, :], 6)
    meta_ref[...] = jnp.concatenate([te, used], axis=1)


def _gmm_body(meta_ref, xg_ref, ws_ref, wg_ref, wu_ref, wd_ref, y_ref):
    j = pl.program_id(0)

    @pl.when(j < meta_ref[NT])
    def _():
        xt = xg_ref[...]
        g = jnp.dot(xt, wg_ref[0], preferred_element_type=jnp.float32)
        u = jnp.dot(xt, wu_ref[0], preferred_element_type=jnp.float32)
        h = (g * jax.nn.sigmoid(g)) * u * ws_ref[...][:, 0:1]
        y_ref[...] = jnp.dot(h, wd_ref[0], preferred_element_type=jnp.float32)


def _sc_scatter_body(x_hbm, slots_hbm, wb_hbm, xg_hbm, ws_hbm, idx_a, idx_b,
                     rows_v, wrows_v, sem0, sem1, sem2, sem3):
    wid = lax.axis_index("s") * 2 + lax.axis_index("c")
    base = wid * BPW
    HB = BPW // 2
    tb = jnp.where(base >= T, base - T, base)
    c0 = pltpu.async_copy(slots_hbm.at[pl.ds(base, HB)], idx_a, sem0)
    c0b = pltpu.async_copy(slots_hbm.at[pl.ds(base + HB, HB)], idx_b, sem3)
    c1 = pltpu.async_copy(x_hbm.at[pl.ds(tb, BPW)], rows_v, sem1)
    c2 = pltpu.async_copy(wb_hbm.at[pl.ds(base, BPW)], wrows_v, sem2)
    c0.wait()
    c0b.wait()
    c1.wait()
    c2.wait()
    c3 = pltpu.async_copy(rows_v.at[pl.ds(0, HB)], xg_hbm.at[idx_a], sem0)
    c4 = pltpu.async_copy(rows_v.at[pl.ds(HB, HB)], xg_hbm.at[idx_b], sem1)
    c5 = pltpu.async_copy(wrows_v.at[pl.ds(0, HB)], ws_hbm.at[idx_a], sem2)
    c6 = pltpu.async_copy(wrows_v.at[pl.ds(HB, HB)], ws_hbm.at[idx_b], sem3)
    c3.wait()
    c4.wait()
    c5.wait()
    c6.wait()


def _sc_combine_body(yw_hbm, slots_hbm, out_hbm, i1_v, i2_v, g1, g2,
                     sem0, sem1, sem2, sem3):
    wid = lax.axis_index("s") * 2 + lax.axis_index("c")
    base = wid * TPW
    H = TPW // 2
    c0 = pltpu.async_copy(slots_hbm.at[pl.ds(base, TPW)], i1_v, sem0)
    c1 = pltpu.async_copy(slots_hbm.at[pl.ds(T + base, TPW)], i2_v, sem1)
    c0.wait()
    c1.wait()
    c1a = pltpu.async_copy(yw_hbm.at[i1_v.at[pl.ds(0, H)]], g1.at[pl.ds(0, H)], sem0)
    c2a = pltpu.async_copy(yw_hbm.at[i2_v.at[pl.ds(0, H)]], g2.at[pl.ds(0, H)], sem1)
    c1b = pltpu.async_copy(yw_hbm.at[i1_v.at[pl.ds(H, H)]], g1.at[pl.ds(H, H)], sem2)
    c2b = pltpu.async_copy(yw_hbm.at[i2_v.at[pl.ds(H, H)]], g2.at[pl.ds(H, H)], sem3)
    c1a.wait()
    c2a.wait()

    @pl.loop(0, H)
    def _(r):
        @pl.loop(0, D, step=16)
        def _(c):
            plsc.addupdate(g1.at[r, pl.ds(c, 16)], g2[r, pl.ds(c, 16)])

    c3a = pltpu.async_copy(g1.at[pl.ds(0, H)], out_hbm.at[pl.ds(base, H)], sem0)
    c1b.wait()
    c2b.wait()

    @pl.loop(H, TPW)
    def _(r):
        @pl.loop(0, D, step=16)
        def _(c):
            plsc.addupdate(g1.at[r, pl.ds(c, 16)], g2[r, pl.ds(c, 16)])

    c3b = pltpu.async_copy(g1.at[pl.ds(H, H)], out_hbm.at[pl.ds(base + H, H)], sem1)
    c3a.wait()
    c3b.wait()


@functools.lru_cache(maxsize=None)
def _sc_kernels():
    mesh = plsc.VectorSubcoreMesh(core_axis_name="c", subcore_axis_name="s")
    scatter = pl.kernel(
        _sc_scatter_body,
        mesh=mesh,
        out_type=(
            jax.ShapeDtypeStruct((S, D), jnp.float32),
            jax.ShapeDtypeStruct((S, 128), jnp.float32),
        ),
        scratch_types=[
            pltpu.VMEM((BPW // 2,), jnp.int32),
            pltpu.VMEM((BPW // 2,), jnp.int32),
            pltpu.VMEM((BPW, D), jnp.float32),
            pltpu.VMEM((BPW, 128), jnp.float32),
            pltpu.SemaphoreType.DMA,
            pltpu.SemaphoreType.DMA,
            pltpu.SemaphoreType.DMA,
            pltpu.SemaphoreType.DMA,
        ],
    )
    combine = pl.kernel(
        _sc_combine_body,
        mesh=mesh,
        out_type=jax.ShapeDtypeStruct((T, D), jnp.float32),
        scratch_types=[
            pltpu.VMEM((TPW,), jnp.int32),
            pltpu.VMEM((TPW,), jnp.int32),
            pltpu.VMEM((TPW, D), jnp.float32),
            pltpu.VMEM((TPW, D), jnp.float32),
            pltpu.SemaphoreType.DMA,
            pltpu.SemaphoreType.DMA,
            pltpu.SemaphoreType.DMA,
            pltpu.SemaphoreType.DMA,
        ],
    )
    return scatter, combine


def kernel(x, gate_W, Wg, Wu, Wd):
    slots2d, wb, meta2d = pl.pallas_call(
        _route_body,
        out_shape=(
            jax.ShapeDtypeStruct((2 * T // 128, 128), jnp.int32),
            jax.ShapeDtypeStruct((2 * T, 128), jnp.float32),
            jax.ShapeDtypeStruct((1, NT + 1), jnp.int32),
        ),
    )(x, gate_W)
    slots = slots2d.reshape(2 * T)
    meta = meta2d.reshape(NT + 1)

    _sc_scatter, _sc_combine = _sc_kernels()
    xg, ws = _sc_scatter(x, slots, wb)

    grid_spec = pltpu.PrefetchScalarGridSpec(
        num_scalar_prefetch=1,
        grid=(NT,),
        in_specs=[
            pl.BlockSpec((BM, D), lambda j, m: (jnp.minimum(j, m[NT] - 1), 0)),
            pl.BlockSpec((BM, 128), lambda j, m: (jnp.minimum(j, m[NT] - 1), 0)),
            pl.BlockSpec((1, D, DFF), lambda j, m: (m[j], 0, 0)),
            pl.BlockSpec((1, D, DFF), lambda j, m: (m[j], 0, 0)),
            pl.BlockSpec((1, DFF, D), lambda j, m: (m[j], 0, 0)),
        ],
        out_specs=pl.BlockSpec((BM, D), lambda j, m: (jnp.minimum(j, m[NT] - 1), 0)),
    )
    yw = pl.pallas_call(
        _gmm_body,
        grid_spec=grid_spec,
        out_shape=jax.ShapeDtypeStruct((S, D), jnp.float32),
    )(meta, xg, ws, Wg, Wu, Wd)

    return _sc_combine(yw, slots)


# speedup vs baseline: 1.2174x; 1.2174x over previous
import functools

import jax
import jax.numpy as jnp
from jax import lax
from jax.experimental import pallas as pl
from jax.experimental.pallas import tpu as pltpu
from jax.experimental.pallas import tpu_sc as plsc

E = 64
TOPK = 2
D = 768
DFF = 768
T = 2048

BM = 128
S = 12288
NT = S // BM
NW = 32
BPW = 2 * T // NW
TPW = T // NW


def _csum_sublane(a, n):
    s = 1
    while s < n:
        z = jnp.zeros((s, a.shape[1]), a.dtype)
        a = a + jnp.concatenate([z, a[: n - s]], axis=0)
        s *= 2
    return a


def _route_body(x_ref, gw_ref, slots_ref, wb_ref, meta_ref):
    logits = jnp.dot(x_ref[...], gw_ref[...], preferred_element_type=jnp.float32)
    e_iota = lax.broadcasted_iota(jnp.int32, (T, E), 1)
    m1 = jnp.max(logits, axis=1, keepdims=True)
    a1 = jnp.min(jnp.where(logits == m1, e_iota, E), axis=1, keepdims=True)
    sel1 = e_iota == a1
    oh1 = sel1.astype(jnp.float32)
    masked = jnp.where(sel1, -jnp.inf, logits)
    m2 = jnp.max(masked, axis=1, keepdims=True)
    a2 = jnp.min(jnp.where(masked == m2, e_iota, E), axis=1, keepdims=True)
    oh2 = (e_iota == a2).astype(jnp.float32)
    w1 = 1.0 / (1.0 + jnp.exp(m2 - m1))
    w2 = 1.0 - w1

    csum12 = _csum_sublane(jnp.concatenate([oh1, oh2], axis=1), T)
    csum1 = csum12[:, :E]
    csum2 = csum12[:, E:]
    cnt1 = csum1[T - 1 : T, :]
    cnt2 = csum2[T - 1 : T, :]
    rank1 = jnp.sum((csum1 - oh1) * oh1, axis=1, keepdims=True)
    rank2 = jnp.sum((cnt1 + csum2 - oh2) * oh2, axis=1, keepdims=True)

    cnt_row = (cnt1 + cnt2).astype(jnp.int32)
    ee_r = lax.broadcasted_iota(jnp.int32, (E, E), 0)
    ee_c = lax.broadcasted_iota(jnp.int32, (E, E), 1)
    ident = (ee_r == ee_c).astype(jnp.int32)
    cnt_col = jnp.sum(ident * cnt_row, axis=1, keepdims=True)
    padded_col = jnp.bitwise_and(cnt_col + (BM - 1), -BM)
    incl_col = _csum_sublane(padded_col, E)
    off_col = incl_col - padded_col
    off_row = jnp.sum(ident.astype(jnp.float32) * off_col.astype(jnp.float32),
                      axis=0, keepdims=True)

    slot1 = jnp.sum(oh1 * off_row, axis=1, keepdims=True) + rank1
    slot2 = jnp.sum(oh2 * off_row, axis=1, keepdims=True) + rank2
    slots_col = jnp.concatenate([slot1, slot2], axis=0).astype(jnp.int32)
    slots_ref[...] = jnp.reshape(slots_col, (2 * T // 128, 128))
    wb_ref[...] = jnp.broadcast_to(jnp.concatenate([w1, w2], axis=0), (2 * T, 128))

    jm = lax.broadcasted_iota(jnp.int32, (E, NT), 1) * BM
    te = jnp.sum((incl_col <= jm).astype(jnp.int32), axis=0, keepdims=True)
    te = jnp.minimum(te, E - 1)
    used = lax.shift_right_logical(incl_col[E - 1 : E, :],
                                   BM.bit_length() - 1)
    meta_ref[...] = jnp.concatenate([te, used], axis=1)


def _gmm_body(meta_ref, xg_ref, ws_ref, wg_ref, wu_ref, wd_ref, y_ref):
    j = pl.program_id(0)

    @pl.when(j < meta_ref[NT])
    def _():
        xt = xg_ref[...]
        g = jnp.dot(xt, wg_ref[0], preferred_element_type=jnp.float32)
        u = jnp.dot(xt, wu_ref[0], preferred_element_type=jnp.float32)
        h = (g * jax.nn.sigmoid(g)) * u * ws_ref[...][:, 0:1]
        y_ref[...] = jnp.dot(h, wd_ref[0], preferred_element_type=jnp.float32)


def _sc_scatter_body(x_hbm, slots_hbm, wb_hbm, xg_hbm, ws_hbm, idx_a, idx_b,
                     rows_v, wrows_v, sem0, sem1, sem2, sem3):
    wid = lax.axis_index("s") * 2 + lax.axis_index("c")
    base = wid * BPW
    HB = BPW // 2
    tb = jnp.where(base >= T, base - T, base)
    c0 = pltpu.async_copy(slots_hbm.at[pl.ds(base, HB)], idx_a, sem0)
    c0b = pltpu.async_copy(slots_hbm.at[pl.ds(base + HB, HB)], idx_b, sem3)
    c1 = pltpu.async_copy(x_hbm.at[pl.ds(tb, BPW)], rows_v, sem1)
    c2 = pltpu.async_copy(wb_hbm.at[pl.ds(base, BPW)], wrows_v, sem2)
    c0.wait()
    c0b.wait()
    c1.wait()
    c2.wait()
    c3 = pltpu.async_copy(rows_v.at[pl.ds(0, HB)], xg_hbm.at[idx_a], sem0)
    c4 = pltpu.async_copy(rows_v.at[pl.ds(HB, HB)], xg_hbm.at[idx_b], sem1)
    c5 = pltpu.async_copy(wrows_v.at[pl.ds(0, HB)], ws_hbm.at[idx_a], sem2)
    c6 = pltpu.async_copy(wrows_v.at[pl.ds(HB, HB)], ws_hbm.at[idx_b], sem3)
    c3.wait()
    c4.wait()
    c5.wait()
    c6.wait()


def _sc_combine_body(yw_hbm, slots_hbm, out_hbm, i1_v, i2_v, g1, g2,
                     sem0, sem1, sem2, sem3):
    wid = lax.axis_index("s") * 2 + lax.axis_index("c")
    base = wid * TPW
    H = TPW // 2
    c0 = pltpu.async_copy(slots_hbm.at[pl.ds(base, TPW)], i1_v, sem0)
    c1 = pltpu.async_copy(slots_hbm.at[pl.ds(T + base, TPW)], i2_v, sem1)
    c0.wait()
    c1.wait()
    c1a = pltpu.async_copy(yw_hbm.at[i1_v.at[pl.ds(0, H)]], g1.at[pl.ds(0, H)], sem0)
    c2a = pltpu.async_copy(yw_hbm.at[i2_v.at[pl.ds(0, H)]], g2.at[pl.ds(0, H)], sem1)
    c1b = pltpu.async_copy(yw_hbm.at[i1_v.at[pl.ds(H, H)]], g1.at[pl.ds(H, H)], sem2)
    c2b = pltpu.async_copy(yw_hbm.at[i2_v.at[pl.ds(H, H)]], g2.at[pl.ds(H, H)], sem3)
    c1a.wait()
    c2a.wait()

    @pl.loop(0, H)
    def _(r):
        @pl.loop(0, D, step=16)
        def _(c):
            plsc.addupdate(g1.at[r, pl.ds(c, 16)], g2[r, pl.ds(c, 16)])

    c3a = pltpu.async_copy(g1.at[pl.ds(0, H)], out_hbm.at[pl.ds(base, H)], sem0)
    c1b.wait()
    c2b.wait()

    @pl.loop(H, TPW)
    def _(r):
        @pl.loop(0, D, step=16)
        def _(c):
            plsc.addupdate(g1.at[r, pl.ds(c, 16)], g2[r, pl.ds(c, 16)])

    c3b = pltpu.async_copy(g1.at[pl.ds(H, H)], out_hbm.at[pl.ds(base + H, H)], sem1)
    c3a.wait()
    c3b.wait()


@functools.lru_cache(maxsize=None)
def _sc_kernels():
    mesh = plsc.VectorSubcoreMesh(core_axis_name="c", subcore_axis_name="s")
    scatter = pl.kernel(
        _sc_scatter_body,
        mesh=mesh,
        out_type=(
            jax.ShapeDtypeStruct((S, D), jnp.float32),
            jax.ShapeDtypeStruct((S, 128), jnp.float32),
        ),
        scratch_types=[
            pltpu.VMEM((BPW // 2,), jnp.int32),
            pltpu.VMEM((BPW // 2,), jnp.int32),
            pltpu.VMEM((BPW, D), jnp.float32),
            pltpu.VMEM((BPW, 128), jnp.float32),
            pltpu.SemaphoreType.DMA,
            pltpu.SemaphoreType.DMA,
            pltpu.SemaphoreType.DMA,
            pltpu.SemaphoreType.DMA,
        ],
    )
    combine = pl.kernel(
        _sc_combine_body,
        mesh=mesh,
        out_type=jax.ShapeDtypeStruct((T, D), jnp.float32),
        scratch_types=[
            pltpu.VMEM((TPW,), jnp.int32),
            pltpu.VMEM((TPW,), jnp.int32),
            pltpu.VMEM((TPW, D), jnp.float32),
            pltpu.VMEM((TPW, D), jnp.float32),
            pltpu.SemaphoreType.DMA,
            pltpu.SemaphoreType.DMA,
            pltpu.SemaphoreType.DMA,
            pltpu.SemaphoreType.DMA,
        ],
    )
    return scatter, combine


def kernel(x, gate_W, Wg, Wu, Wd):
    slots2d, wb, meta2d = pl.pallas_call(
        _route_body,
        out_shape=(
            jax.ShapeDtypeStruct((2 * T // 128, 128), jnp.int32),
            jax.ShapeDtypeStruct((2 * T, 128), jnp.float32),
            jax.ShapeDtypeStruct((1, NT + 1), jnp.int32),
        ),
    )(x, gate_W)
    slots = slots2d.reshape(2 * T)
    meta = meta2d.reshape(NT + 1)

    _sc_scatter, _sc_combine = _sc_kernels()
    xg, ws = _sc_scatter(x, slots, wb)

    grid_spec = pltpu.PrefetchScalarGridSpec(
        num_scalar_prefetch=1,
        grid=(NT,),
        in_specs=[
            pl.BlockSpec((BM, D), lambda j, m: (jnp.minimum(j, m[NT] - 1), 0)),
            pl.BlockSpec((BM, 128), lambda j, m: (jnp.minimum(j, m[NT] - 1), 0)),
            pl.BlockSpec((1, D, DFF), lambda j, m: (m[j], 0, 0)),
            pl.BlockSpec((1, D, DFF), lambda j, m: (m[j], 0, 0)),
            pl.BlockSpec((1, DFF, D), lambda j, m: (m[j], 0, 0)),
        ],
        out_specs=pl.BlockSpec((BM, D), lambda j, m: (jnp.minimum(j, m[NT] - 1), 0)),
    )
    yw = pl.pallas_call(
        _gmm_body,
        grid_spec=grid_spec,
        out_shape=jax.ShapeDtypeStruct((S, D), jnp.float32),
    )(meta, xg, ws, Wg, Wu, Wd)

    return _sc_combine(yw, slots)
